# baseline (speedup 1.0000x reference)
import functools
import math

import jax
import jax.numpy as jnp
from jax import lax
from jax.experimental import pallas as pl
from jax.experimental.pallas import tpu as pltpu
from jax.experimental.pallas import tpu_sc as plsc

_EPS = 1e-5


def _sc_geometry():
    try:
        info = plsc.get_sparse_core_info()
        return int(info.num_cores), int(info.num_subcores)
    except Exception:
        return 2, 16


@functools.lru_cache(maxsize=None)
def _make_gather_mean(b_sz, l_real, d, nc, ns):
    n_items = 3 * b_sz
    nw = nc * ns
    assert n_items % nw == 0, (n_items, nw)
    ipw = n_items // nw
    c_items = 4
    while ipw % (2 * c_items) or b_sz % c_items:
        c_items //= 2
    assert c_items >= 1
    nch = ipw // c_items
    n_str = 4
    seg = (l_real // n_str + 7) // 8 * 8
    segs = []
    off = 0
    while off < l_real:
        ln = min(seg, l_real - off)
        segs.append((off, ln))
        off += ln
    assert all(o % 8 == 0 and 0 < ln <= 128 for o, ln in segs)
    assert l_real % 8 == 0
    nj = d // 16
    assert d % 16 == 0

    mesh = plsc.VectorSubcoreMesh(core_axis_name="c", subcore_axis_name="s")

    @functools.partial(
        pl.kernel,
        mesh=mesh,
        compiler_params=pltpu.CompilerParams(use_tc_tiling_on_sc=False),
        out_type=jax.ShapeDtypeStruct((n_items, d), jnp.float32),
        scratch_types=[
            pltpu.VMEM((2, c_items, l_real), jnp.int32),
            pltpu.VMEM((2, c_items, l_real, d), jnp.float32),
            pltpu.VMEM((c_items, d), jnp.float32),
            pltpu.SemaphoreType.DMA,
            pltpu.SemaphoreType.DMA,
        ],
    )
    def gather_mean(p_hbm, q_hbm, a_hbm, table_hbm, out_hbm,
                    idx_v, rows_v, outs_v, sem0, sem1):
        sems = (sem0, sem1)
        wid = lax.axis_index("s") * nc + lax.axis_index("c")
        base = wid * ipw

        def copy_idx(ch, b):
            row = base + ch * c_items
            field = row // b_sz
            local = row - field * b_sz
            for f, src in enumerate((p_hbm, q_hbm, a_hbm)):
                @pl.when(field == f)
                def _cp(src=src):
                    pltpu.sync_copy(src.at[pl.ds(local, c_items)],
                                    idx_v.at[b])

        def gather_copies(b):
            cps = []
            for c in range(c_items):
                for off, ln in segs:
                    cps.append(pltpu.make_async_copy(
                        table_hbm.at[idx_v.at[b, c, pl.ds(off, ln)]],
                        rows_v.at[b, c, pl.ds(off, ln)],
                        sems[b]))
            return cps

        def fire(b):
            for cp in gather_copies(b):
                cp.start()

        def drain(b):
            for cp in gather_copies(b):
                cp.wait()

        def process(ch, b):
            row = base + ch * c_items
            for c in range(c_items):
                def body(it, carry, c=c):
                    accs = list(carry)
                    for rr in range(8):
                        r = it * 8 + rr
                        for j in range(nj):
                            accs[j] = accs[j] + rows_v[b, c, r,
                                                       pl.ds(j * 16, 16)]
                    return tuple(accs)
                accs = lax.fori_loop(
                    0, l_real // 8, body,
                    (jnp.zeros((16,), jnp.float32),) * nj)
                for j in range(nj):
                    outs_v[c, pl.ds(j * 16, 16)] = accs[j] * (1.0 / l_real)
            pltpu.sync_copy(outs_v, out_hbm.at[pl.ds(row, c_items)])

        copy_idx(0, 0)
        fire(0)

        def outer(g, carry):
            for b in range(2):
                ch = g * 2 + b
                nb = 1 - b

                @pl.when(ch + 1 < nch)
                def _fire_next(ch=ch, nb=nb):
                    copy_idx(ch + 1, nb)
                    fire(nb)

                drain(b)
                process(ch, b)
            return carry

        lax.fori_loop(0, nch // 2, outer, 0)

    return gather_mean


@functools.lru_cache(maxsize=None)
def _make_mlp(batch, rows, f_dim, e_dim, hidden):
    assert batch % rows == 0
    nblk = batch // rows

    def body(fin, p, q, a, w0f, w0p, w0q, w0a, b0, w1, b1, w2, b2, w3, b3,
             wo, bo, out):
        h = jnp.dot(fin[...], w0f[...], preferred_element_type=jnp.float32)
        h = h + jnp.dot(p[...], w0p[...], preferred_element_type=jnp.float32)
        h = h + jnp.dot(q[...], w0q[...], preferred_element_type=jnp.float32)
        h = h + jnp.dot(a[...], w0a[...], preferred_element_type=jnp.float32)
        h = jnp.maximum(h + b0[...], 0.0)
        for w, bb in ((w1, b1), (w2, b2), (w3, b3)):
            h = jnp.dot(h, w[...], preferred_element_type=jnp.float32)
            h = jnp.maximum(h + bb[...], 0.0)
        out[...] = jnp.sum(h * wo[...], axis=1, keepdims=True) + bo[...]

    def full(shape):
        return pl.BlockSpec(shape, lambda i: (0,) * len(shape))

    def emb_spec(f):
        return pl.BlockSpec((rows, e_dim), lambda i, f=f: (f * nblk + i, 0))

    return pl.pallas_call(
        body,
        grid=(nblk,),
        in_specs=[
            pl.BlockSpec((rows, f_dim), lambda i: (i, 0)),
            emb_spec(0), emb_spec(1), emb_spec(2),
            full((f_dim, hidden)),
            full((e_dim, hidden)), full((e_dim, hidden)), full((e_dim, hidden)),
            full((1, hidden)),
            full((hidden, hidden)), full((1, hidden)),
            full((hidden, hidden)), full((1, hidden)),
            full((hidden, hidden)), full((1, hidden)),
            full((1, hidden)), full((1, 1)),
        ],
        out_specs=pl.BlockSpec((rows, 1), lambda i: (i, 0)),
        out_shape=jax.ShapeDtypeStruct((batch, 1), jnp.float32),
    )


def kernel(finance_features, presentation_toks_np, question_1_toks_np,
           answer_1_toks_np, table, W0, b0, W1, b1, W2, b2, W3, b3,
           Wout, bout):
    b_sz, l_tok = presentation_toks_np.shape
    v_sz, d = table.shape
    f_dim = finance_features.shape[1]
    hidden = W0.shape[0]

    p_toks = presentation_toks_np.astype(jnp.int32)
    q_toks = question_1_toks_np.astype(jnp.int32)
    a_toks = answer_1_toks_np.astype(jnp.int32)

    nc, ns = _sc_geometry()
    means = _make_gather_mean(b_sz, l_tok, d, nc, ns)(
        p_toks, q_toks, a_toks, table)

    inv = jnp.float32(1.0 / math.sqrt(1.0 + _EPS))
    w0 = W0 * inv
    out = _make_mlp(b_sz, 1024, f_dim, d, hidden)(
        finance_features, means, means, means,
        w0[:, :f_dim].T,
        w0[:, f_dim:f_dim + d].T,
        w0[:, f_dim + d:f_dim + 2 * d].T,
        w0[:, f_dim + 2 * d:].T,
        b0.reshape(1, hidden),
        (W1 * inv).T, b1.reshape(1, hidden),
        (W2 * inv).T, b2.reshape(1, hidden),
        (W3 * inv).T, b3.reshape(1, hidden),
        Wout * inv, bout.reshape(1, 1))
    return out


# speedup vs baseline: 1.0087x; 1.0087x over previous
import functools
import math

import jax
import jax.numpy as jnp
from jax import lax
from jax.experimental import pallas as pl
from jax.experimental.pallas import tpu as pltpu
from jax.experimental.pallas import tpu_sc as plsc

_EPS = 1e-5


def _sc_geometry():
    try:
        info = plsc.get_sparse_core_info()
        return int(info.num_cores), int(info.num_subcores)
    except Exception:
        return 2, 16


@functools.lru_cache(maxsize=None)
def _make_gather_mean(b_sz, l_real, d, nc, ns):
    n_items = 3 * b_sz
    nw = nc * ns
    assert n_items % nw == 0, (n_items, nw)
    ipw = n_items // nw
    c_items = 4
    while ipw % (2 * c_items) or b_sz % c_items:
        c_items //= 2
    assert c_items >= 1
    nch = ipw // c_items
    n_str = 4
    seg = (l_real // n_str + 7) // 8 * 8
    segs = []
    off = 0
    while off < l_real:
        ln = min(seg, l_real - off)
        segs.append((off, ln))
        off += ln
    assert all(o % 8 == 0 and 0 < ln <= 128 for o, ln in segs)
    assert l_real % 8 == 0
    nj = d // 16
    assert d % 16 == 0

    mesh = plsc.VectorSubcoreMesh(core_axis_name="c", subcore_axis_name="s")

    @functools.partial(
        pl.kernel,
        mesh=mesh,
        compiler_params=pltpu.CompilerParams(use_tc_tiling_on_sc=False),
        out_type=jax.ShapeDtypeStruct((n_items, d), jnp.float32),
        scratch_types=[
            pltpu.VMEM((2, c_items, l_real), jnp.int32),
            pltpu.VMEM((2, c_items, l_real, d), jnp.float32),
            pltpu.VMEM((c_items, d), jnp.float32),
            pltpu.SemaphoreType.DMA,
            pltpu.SemaphoreType.DMA,
        ],
    )
    def gather_mean(p_hbm, q_hbm, a_hbm, table_hbm, out_hbm,
                    idx_v, rows_v, outs_v, sem0, sem1):
        sems = (sem0, sem1)
        wid = lax.axis_index("s") * nc + lax.axis_index("c")
        base = wid * ipw

        def copy_idx(ch, b):
            row = base + ch * c_items
            field = row // b_sz
            local = row - field * b_sz
            for f, src in enumerate((p_hbm, q_hbm, a_hbm)):
                @pl.when(field == f)
                def _cp(src=src):
                    pltpu.sync_copy(src.at[pl.ds(local, c_items)],
                                    idx_v.at[b])

        def gather_copies(b):
            cps = []
            for c in range(c_items):
                for off, ln in segs:
                    cps.append(pltpu.make_async_copy(
                        table_hbm.at[idx_v.at[b, c, pl.ds(off, ln)]],
                        rows_v.at[b, c, pl.ds(off, ln)],
                        sems[b]))
            return cps

        def fire(b):
            for cp in gather_copies(b):
                cp.start()

        def drain(b):
            for cp in gather_copies(b):
                cp.wait()

        def process(ch, b):
            row = base + ch * c_items
            for c in range(c_items):
                def body(it, carry, c=c):
                    accs = list(carry)
                    for rr in range(8):
                        r = it * 8 + rr
                        for j in range(nj):
                            accs[j] = accs[j] + rows_v[b, c, r,
                                                       pl.ds(j * 16, 16)]
                    return tuple(accs)
                accs = lax.fori_loop(
                    0, l_real // 8, body,
                    (jnp.zeros((16,), jnp.float32),) * nj)
                for j in range(nj):
                    outs_v[c, pl.ds(j * 16, 16)] = accs[j] * (1.0 / l_real)
            pltpu.sync_copy(outs_v, out_hbm.at[pl.ds(row, c_items)])

        copy_idx(0, 0)
        fire(0)

        def outer(g, carry):
            for b in range(2):
                ch = g * 2 + b
                nb = 1 - b

                @pl.when(ch + 1 < nch)
                def _fire_next(ch=ch, nb=nb):
                    copy_idx(ch + 1, nb)
                    fire(nb)

                drain(b)
                process(ch, b)
            return carry

        lax.fori_loop(0, nch // 2, outer, 0)

    return gather_mean


@functools.lru_cache(maxsize=None)
def _make_mlp(batch, rows, f_dim, e_dim, hidden):
    assert batch % rows == 0
    nblk = batch // rows

    def body(fin, p, q, a, w0f, w0p, w0q, w0a, b0, w1, b1, w2, b2, w3, b3,
             wo, bo, out):
        h = jnp.dot(fin[...], w0f[...], preferred_element_type=jnp.float32)
        h = h + jnp.dot(p[...], w0p[...], preferred_element_type=jnp.float32)
        h = h + jnp.dot(q[...], w0q[...], preferred_element_type=jnp.float32)
        h = h + jnp.dot(a[...], w0a[...], preferred_element_type=jnp.float32)
        h = jnp.maximum(h + b0[...], 0.0)
        for w, bb in ((w1, b1), (w2, b2), (w3, b3)):
            h = jnp.dot(h, w[...], preferred_element_type=jnp.float32)
            h = jnp.maximum(h + bb[...], 0.0)
        out[...] = jnp.sum(h * wo[...], axis=1, keepdims=True) + bo[...]

    def full(shape):
        return pl.BlockSpec(shape, lambda i: (0,) * len(shape))

    def emb_spec(f):
        return pl.BlockSpec((rows, e_dim), lambda i, f=f: (f * nblk + i, 0))

    return pl.pallas_call(
        body,
        grid=(nblk,),
        in_specs=[
            pl.BlockSpec((rows, f_dim), lambda i: (i, 0)),
            emb_spec(0), emb_spec(1), emb_spec(2),
            full((f_dim, hidden)),
            full((e_dim, hidden)), full((e_dim, hidden)), full((e_dim, hidden)),
            full((1, hidden)),
            full((hidden, hidden)), full((1, hidden)),
            full((hidden, hidden)), full((1, hidden)),
            full((hidden, hidden)), full((1, hidden)),
            full((1, hidden)), full((1, 1)),
        ],
        out_specs=pl.BlockSpec((rows, 1), lambda i: (i, 0)),
        out_shape=jax.ShapeDtypeStruct((batch, 1), jnp.float32),
    )


def kernel(finance_features, presentation_toks_np, question_1_toks_np,
           answer_1_toks_np, table, W0, b0, W1, b1, W2, b2, W3, b3,
           Wout, bout):
    b_sz, l_tok = presentation_toks_np.shape
    v_sz, d = table.shape
    f_dim = finance_features.shape[1]
    hidden = W0.shape[0]

    p_toks = presentation_toks_np.astype(jnp.int32)
    q_toks = question_1_toks_np.astype(jnp.int32)
    a_toks = answer_1_toks_np.astype(jnp.int32)

    nc, ns = _sc_geometry()

    inv = jnp.float32(1.0 / math.sqrt(1.0 + _EPS))
    w0 = W0 * inv
    mlp_args = (
        w0[:, :f_dim].T,
        w0[:, f_dim:f_dim + d].T,
        w0[:, f_dim + d:f_dim + 2 * d].T,
        w0[:, f_dim + 2 * d:].T,
        b0.reshape(1, hidden),
        (W1 * inv).T, b1.reshape(1, hidden),
        (W2 * inv).T, b2.reshape(1, hidden),
        (W3 * inv).T, b3.reshape(1, hidden),
        Wout * inv, bout.reshape(1, 1))

    half = b_sz // 2
    if b_sz % 2 == 0 and (3 * half) % (nc * ns) == 0 and half % 1024 == 0:
        sc = _make_gather_mean(half, l_tok, d, nc, ns)
        mlp = _make_mlp(half, 1024, f_dim, d, hidden)
        outs = []
        for lo in (0, half):
            means = sc(p_toks[lo:lo + half], q_toks[lo:lo + half],
                       a_toks[lo:lo + half], table)
            outs.append(mlp(finance_features[lo:lo + half],
                            means, means, means, *mlp_args))
        return jnp.concatenate(outs, axis=0)

    means = _make_gather_mean(b_sz, l_tok, d, nc, ns)(
        p_toks, q_toks, a_toks, table)
    out = _make_mlp(b_sz, 1024, f_dim, d, hidden)(
        finance_features, means, means, means, *mlp_args)
    return out
